# baseline (speedup 1.0000x reference)
import jax
import jax.numpy as jnp
from jax.experimental import pallas as pl
from jax.experimental.pallas import tpu as pltpu

_CHUNKS = 8


def _compgcn_kernel(adj_ref, x_ref, rel_ref, ws_ref, wl_ref, wr_ref,
                    b_ref, lr_ref, out_ref, mask_ref, deg_ref):
    i = pl.program_id(0)
    n = x_ref.shape[0]
    r_count = rel_ref.shape[0]
    num_layers = ws_ref.shape[0]
    rows = adj_ref.shape[0]
    f32 = jnp.float32
    dn_t = (((0,), (0,)), ((), ()))
    dn = (((1,), (0,)), ((), ()))

    m = (adj_ref[...] > 0.5).astype(f32)
    mask_ref[pl.ds(i * rows, rows), :] = m
    part = jax.lax.dot_general(m, jnp.ones((rows, 1), f32), dn_t)

    @pl.when(i == 0)
    def _init():
        deg_ref[...] = part

    @pl.when(i > 0)
    def _acc():
        deg_ref[...] = deg_ref[...] + part

    @pl.when(i == _CHUNKS - 1)
    def _layers():
        deg = deg_ref[...]
        norm = jnp.where(deg > 0.0, jax.lax.rsqrt(deg), 0.0)
        h = x_ref[...]
        r = rel_ref[...]
        for l in range(num_layers):
            hn = h * norm
            p = jnp.zeros_like(h)
            for et in range(r_count):
                comp = hn * r[et:et + 1, :]
                p = p + jax.lax.dot_general(
                    mask_ref[pl.ds(et * n, n), :], comp, dn_t)
            agg = jax.lax.dot_general(p, ws_ref[l], dn) * norm
            loop = jax.lax.dot_general(h * lr_ref[l], wl_ref[l], dn)
            h = jnp.tanh(agg + loop + b_ref[l:l + 1, :])
            if l + 1 < num_layers:
                r = jax.lax.dot_general(r, wr_ref[l], dn)
        out_ref[...] = h


@jax.jit
def kernel(x, fw_adjs, init_rel, Ws, W_loops, W_rels, biases, loop_rels):
    n, h_dim = x.shape
    r_count = fw_adjs.shape[0]
    num_layers = Ws.shape[0]
    rel = init_rel[:r_count]
    wr = W_rels[:num_layers - 1]
    adj = fw_adjs.reshape(r_count * n, n)
    chunk = (r_count * n) // _CHUNKS
    full = lambda i: (0, 0)
    full3 = lambda i: (0, 0, 0)
    return pl.pallas_call(
        _compgcn_kernel,
        grid=(_CHUNKS,),
        in_specs=[
            pl.BlockSpec((chunk, n), lambda i: (i, 0)),
            pl.BlockSpec((n, h_dim), full),
            pl.BlockSpec(rel.shape, full),
            pl.BlockSpec(Ws.shape, full3),
            pl.BlockSpec(W_loops.shape, full3),
            pl.BlockSpec(wr.shape, full3),
            pl.BlockSpec(biases.shape, full),
            pl.BlockSpec(loop_rels.shape, full3),
        ],
        out_specs=pl.BlockSpec((n, h_dim), full),
        out_shape=jax.ShapeDtypeStruct((n, h_dim), x.dtype),
        scratch_shapes=[
            pltpu.VMEM((r_count * n, n), jnp.float32),
            pltpu.VMEM((n, 1), jnp.float32),
        ],
    )(adj, x, rel, Ws, W_loops, wr, biases, loop_rels)


# speedup vs baseline: 1.4576x; 1.4576x over previous
import jax
import jax.numpy as jnp
from jax.experimental import pallas as pl


def _compgcn_kernel(adj_ref, x_ref, rel_ref, ws_ref, wl_ref, wr_ref,
                    b_ref, lr_ref, out_ref):
    n = x_ref.shape[0]
    r_count = adj_ref.shape[0]
    num_layers = ws_ref.shape[0]
    f32 = jnp.float32
    dn_t = (((0,), (0,)), ((), ()))
    dn = (((1,), (0,)), ((), ()))

    ones_col = jnp.ones((n, 1), f32)
    deg = jnp.zeros((n, 1), f32)
    masks = []
    for et in range(r_count):
        m = (adj_ref[et] > 0.5).astype(f32)
        masks.append(m)
        deg = deg + jax.lax.dot_general(m, ones_col, dn_t)
    norm = jnp.where(deg > 0.0, jax.lax.rsqrt(deg), 0.0)

    h = x_ref[...]
    r = rel_ref[...]
    for l in range(num_layers):
        hn = h * norm
        p = jnp.zeros_like(h)
        for et in range(r_count):
            comp = hn * r[et:et + 1, :]
            p = p + jax.lax.dot_general(masks[et], comp, dn_t)
        agg = jax.lax.dot_general(p, ws_ref[l], dn) * norm
        loop = jax.lax.dot_general(h * lr_ref[l], wl_ref[l], dn)
        h = jnp.tanh(agg + loop + b_ref[l:l + 1, :])
        if l + 1 < num_layers:
            r = jax.lax.dot_general(r, wr_ref[l], dn)
    out_ref[...] = h


@jax.jit
def kernel(x, fw_adjs, init_rel, Ws, W_loops, W_rels, biases, loop_rels):
    n, h_dim = x.shape
    r_count = fw_adjs.shape[0]
    num_layers = Ws.shape[0]
    rel = init_rel[:r_count]
    wr = W_rels[:num_layers - 1]
    return pl.pallas_call(
        _compgcn_kernel,
        out_shape=jax.ShapeDtypeStruct((n, h_dim), x.dtype),
    )(fw_adjs, x, rel, Ws, W_loops, wr, biases, loop_rels)


# speedup vs baseline: 2.2595x; 1.5501x over previous
import jax
import jax.numpy as jnp
from jax.experimental import pallas as pl


def _compgcn_kernel(adj_ref, x_ref, rel_ref, ws_ref, wl_ref, wr_ref,
                    b_ref, lr_ref, out_ref):
    n = x_ref.shape[0]
    r_count = adj_ref.shape[0]
    num_layers = ws_ref.shape[0]
    f32 = jnp.float32
    dn_t = (((0,), (0,)), ((), ()))
    dn = (((1,), (0,)), ((), ()))

    ones_col = jnp.ones((n, 1), f32)
    deg = jnp.zeros((n, 1), f32)
    masks = []
    for et in range(r_count):
        m = (adj_ref[et] > 0.5).astype(f32)
        masks.append(m)
        deg = deg + jax.lax.dot_general(m, ones_col, dn_t)
    norm = jnp.where(deg > 0.0, jax.lax.rsqrt(deg), 0.0)

    h = x_ref[...]
    r = rel_ref[0:r_count, :]
    for l in range(num_layers):
        hn = h * norm
        p = jnp.zeros_like(h)
        for et in range(r_count):
            comp = hn * r[et:et + 1, :]
            p = p + jax.lax.dot_general(masks[et], comp, dn_t)
        agg = jax.lax.dot_general(p, ws_ref[l], dn) * norm
        loop = jax.lax.dot_general(h * lr_ref[l], wl_ref[l], dn)
        h = jnp.tanh(agg + loop + b_ref[l:l + 1, :])
        if l + 1 < num_layers:
            r = jax.lax.dot_general(r, wr_ref[l], dn)
    out_ref[...] = h


@jax.jit
def kernel(x, fw_adjs, init_rel, Ws, W_loops, W_rels, biases, loop_rels):
    n, h_dim = x.shape
    return pl.pallas_call(
        _compgcn_kernel,
        out_shape=jax.ShapeDtypeStruct((n, h_dim), x.dtype),
    )(fw_adjs, x, init_rel, Ws, W_loops, W_rels, biases, loop_rels)


# speedup vs baseline: 2.2653x; 1.0026x over previous
import jax
import jax.numpy as jnp
from jax.experimental import pallas as pl


def _compgcn_kernel(adj_ref, x_ref, rel_ref, ws_ref, wl_ref, wr_ref,
                    b_ref, lr_ref, out_ref):
    n = x_ref.shape[0]
    r_count = adj_ref.shape[0]
    num_layers = ws_ref.shape[0]
    f32 = jnp.float32
    dn_t = (((0,), (0,)), ((), ()))
    dn = (((1,), (0,)), ((), ()))

    bf16 = jnp.bfloat16
    mcat = jnp.concatenate(
        [(adj_ref[et] > 0.5).astype(bf16) for et in range(r_count)], axis=0)
    deg = jax.lax.dot_general(mcat, jnp.ones((r_count * n, 1), bf16), dn_t,
                              preferred_element_type=f32)
    norm = jnp.where(deg > 0.0, jax.lax.rsqrt(deg), 0.0)

    h = x_ref[...]
    r = rel_ref[0:r_count, :]
    for l in range(num_layers):
        hn = h * norm
        comp = jnp.concatenate(
            [(hn * r[et:et + 1, :]).astype(bf16) for et in range(r_count)],
            axis=0)
        p = jax.lax.dot_general(mcat, comp, dn_t,
                                preferred_element_type=f32)
        agg = jax.lax.dot_general(p, ws_ref[l], dn) * norm
        loop = jax.lax.dot_general(h * lr_ref[l], wl_ref[l], dn)
        h = jnp.tanh(agg + loop + b_ref[l:l + 1, :])
        if l + 1 < num_layers:
            r = jax.lax.dot_general(r, wr_ref[l], dn)
    out_ref[...] = h


@jax.jit
def kernel(x, fw_adjs, init_rel, Ws, W_loops, W_rels, biases, loop_rels):
    n, h_dim = x.shape
    return pl.pallas_call(
        _compgcn_kernel,
        out_shape=jax.ShapeDtypeStruct((n, h_dim), x.dtype),
    )(fw_adjs, x, init_rel, Ws, W_loops, W_rels, biases, loop_rels)


# speedup vs baseline: 2.6299x; 1.1609x over previous
import jax
import jax.numpy as jnp
from jax.experimental import pallas as pl


def _compgcn_kernel(adj_ref, x_ref, rel_ref, ws_ref, wl_ref, wr_ref,
                    b_ref, lr_ref, out_ref):
    n = x_ref.shape[0]
    r_count = adj_ref.shape[0]
    num_layers = ws_ref.shape[0]
    f32 = jnp.float32
    bf16 = jnp.bfloat16
    dn = (((1,), (0,)), ((), ()))
    dn_t = (((0,), (0,)), ((), ()))

    mcat = jnp.concatenate(
        [(adj_ref[et] > 0.5).astype(bf16) for et in range(r_count)], axis=0)
    deg = jax.lax.dot_general(jnp.ones((1, r_count * n), bf16), mcat, dn,
                              preferred_element_type=f32)
    norm = jnp.where(deg > 0.0, jax.lax.rsqrt(deg), 0.0)

    ht = x_ref[...].T
    rt = rel_ref[0:r_count, :].T
    bt = b_ref[...].T
    lrt = jnp.concatenate([lr_ref[l] for l in range(num_layers)], axis=0).T
    for l in range(num_layers):
        hnt = ht * norm
        compt = jnp.concatenate(
            [(hnt * rt[:, et:et + 1]).astype(bf16) for et in range(r_count)],
            axis=1)
        pt = jax.lax.dot_general(compt, mcat, dn,
                                 preferred_element_type=f32)
        aggt = jax.lax.dot_general(ws_ref[l], pt, dn_t) * norm
        loopt = jax.lax.dot_general(wl_ref[l], ht * lrt[:, l:l + 1], dn_t)
        ht = jnp.tanh(aggt + loopt + bt[:, l:l + 1])
        if l + 1 < num_layers:
            rt = jax.lax.dot_general(wr_ref[l], rt, dn_t)
    out_ref[...] = ht.T


@jax.jit
def kernel(x, fw_adjs, init_rel, Ws, W_loops, W_rels, biases, loop_rels):
    n, h_dim = x.shape
    return pl.pallas_call(
        _compgcn_kernel,
        out_shape=jax.ShapeDtypeStruct((n, h_dim), x.dtype),
    )(fw_adjs, x, init_rel, Ws, W_loops, W_rels, biases, loop_rels)


# speedup vs baseline: 2.6486x; 1.0071x over previous
import jax
import jax.numpy as jnp
from jax.experimental import pallas as pl


def _compgcn_kernel(adj_ref, x_ref, rel_ref, ws_ref, wl_ref, wr_ref,
                    b_ref, lr_ref, out_ref):
    n = x_ref.shape[0]
    r_count = adj_ref.shape[0]
    num_layers = ws_ref.shape[0]
    f32 = jnp.float32
    bf16 = jnp.bfloat16
    dn = (((1,), (0,)), ((), ()))
    dn_t = (((0,), (0,)), ((), ()))

    mcat = jnp.concatenate(
        [(adj_ref[et] > 0.5).astype(bf16) for et in range(r_count)], axis=0)
    deg = jax.lax.dot_general(jnp.ones((1, r_count * n), bf16), mcat, dn,
                              preferred_element_type=f32)
    norm = jnp.where(deg > 0.0, jax.lax.rsqrt(deg), 0.0)
    normcat = jnp.concatenate([norm] * r_count, axis=1)

    ht = x_ref[...].T
    rt = rel_ref[0:r_count, :].T
    bt = b_ref[...].T
    lrt = jnp.concatenate([lr_ref[l] for l in range(num_layers)], axis=0).T
    for l in range(num_layers):
        comp = jnp.concatenate(
            [ht * rt[:, et:et + 1] for et in range(r_count)], axis=1)
        compw = jax.lax.dot_general(ws_ref[l], comp, dn_t)
        compw = (compw * normcat).astype(bf16)
        aggt = jax.lax.dot_general(compw, mcat, dn,
                                   preferred_element_type=f32) * norm
        loopt = jax.lax.dot_general(wl_ref[l], ht * lrt[:, l:l + 1], dn_t)
        ht = jnp.tanh(aggt + loopt + bt[:, l:l + 1])
        if l + 1 < num_layers:
            rt = jax.lax.dot_general(wr_ref[l], rt, dn_t)
    out_ref[...] = ht.T


@jax.jit
def kernel(x, fw_adjs, init_rel, Ws, W_loops, W_rels, biases, loop_rels):
    n, h_dim = x.shape
    return pl.pallas_call(
        _compgcn_kernel,
        out_shape=jax.ShapeDtypeStruct((n, h_dim), x.dtype),
    )(fw_adjs, x, init_rel, Ws, W_loops, W_rels, biases, loop_rels)
